# baseline (speedup 1.0000x reference)
import math

import jax
import jax.numpy as jnp
from jax import lax
from jax.experimental import pallas as pl
from jax.experimental.pallas import tpu as pltpu
from jax.experimental.pallas import tpu_sc as plsc

_VOCAB = 100000
_PAD_IDX = 0
_SMOOTH = 0.1
_CONF = 1.0 - _SMOOTH
_N_TOK = 2048
_SVAL = _SMOOTH / (_VOCAB - 2)
_E_TERM = (_VOCAB - 2) * _SVAL * math.log(_SVAL) + _CONF * math.log(_CONF)

_CS = 38400
_W = 1920
_NCH = _CS // _W

_RB = 256
_VB = 12800
_RGRID = _N_TOK // _RB
_VGRID = -(-_VOCAB // _VB)
_JGRID = _VGRID - _CS // _VB

_NW = 32
_RW = _N_TOK // _NW
_LANES = 16
_TILE_S = 8
_TILE_L = 128

_TAIL_BASE = (_VGRID - 1) * _VB
_TAIL_FULL = (_VOCAB - _TAIL_BASE) // 128
_TAIL_REM = _VOCAB - _TAIL_BASE - _TAIL_FULL * 128


def _psum_lanes(y_ref, nslices, init=None):
    acc = init
    for k in range(nslices):
        s = y_ref[:, k * 128:(k + 1) * 128]
        acc = s if acc is None else acc + s
    return acc


def _rowsum_body(y_ref, tgt_ref, out_ref, acc_ref):
    i = pl.program_id(0)
    j = pl.program_id(1)

    @pl.when(j == 0)
    def _init():
        acc_ref[...] = _psum_lanes(y_ref, _VB // 128)

    @pl.when(jnp.logical_and(j > 0, j < _JGRID - 1))
    def _acc():
        acc_ref[...] = _psum_lanes(y_ref, _VB // 128, acc_ref[...])

    @pl.when(j == _JGRID - 1)
    def _acc_tail():
        acc = _psum_lanes(y_ref, _TAIL_FULL, acc_ref[...])
        if _TAIL_REM:
            lanes = lax.broadcasted_iota(jnp.int32, (_RB, 128), 1)
            part = y_ref[:, _TAIL_FULL * 128:(_TAIL_FULL + 1) * 128]
            acc = acc + jnp.where(lanes < _TAIL_REM, part, 0.0)
        rs_row = jnp.sum(acc, axis=1, keepdims=True)
        masked = jnp.where(tgt_ref[...] != _PAD_IDX, rs_row, 0.0)
        partial = jnp.sum(masked)

        @pl.when(i == 0)
        def _first():
            out_ref[0, 0] = partial

        @pl.when(i > 0)
        def _rest():
            out_ref[0, 0] += partial


def _tc_masked_rowsum(y, target_2d):
    return pl.pallas_call(
        _rowsum_body,
        grid=(_RGRID, _JGRID),
        in_specs=[
            pl.BlockSpec((_RB, _VB), lambda i, j: (i, j + _CS // _VB)),
            pl.BlockSpec((_RB, 1), lambda i, j: (i, 0)),
        ],
        out_specs=pl.BlockSpec((1, 1), lambda i, j: (0, 0),
                               memory_space=pltpu.SMEM),
        out_shape=jax.ShapeDtypeStruct((1, 1), jnp.float32),
        scratch_shapes=[pltpu.VMEM((_RB, 128), jnp.float32)],
    )(y, target_2d)


def _sc_body(y_hbm, tgt_hbm, out_hbm,
             tgt_v, buf_t, buf_0, rsb_v, buf_a, buf_b, acc_v,
             sem, sem2, sem_a, sem_b):
    wid = lax.axis_index("s") * 2 + lax.axis_index("c")
    base = wid * _RW
    pltpu.sync_copy(tgt_hbm.at[pl.ds(base, _RW)], tgt_v)
    col0 = pltpu.async_copy(
        y_hbm.at[pl.ds(base, _RW), pl.ds(0, _TILE_L)], buf_0, sem2)
    iota16 = lax.iota(jnp.int32, _LANES)
    copies = []
    for r in range(_RW):
        t16 = tgt_v[pl.ds((r // _LANES) * _LANES, _LANES)]
        t = jnp.sum(jnp.where(iota16 == (r % _LANES), t16, 0), axis=0)
        cb = pl.multiple_of((t // _TILE_L) * _TILE_L, _TILE_L)
        rg = pl.multiple_of(base + (r // _TILE_S) * _TILE_S, _TILE_S)
        copies.append(pltpu.async_copy(
            y_hbm.at[pl.ds(rg, _TILE_S), pl.ds(cb, _TILE_L)],
            buf_t.at[r], sem))

    def _drain_accum_refill(buf, dsem, accs, rows, nxt):
        pltpu.make_async_copy(
            y_hbm.at[pl.ds(0, _TILE_S), pl.ds(0, _W)], buf, dsem).wait()

        @pl.loop(0, _W // _LANES, init_carry=accs, unroll=8)
        def inner(s, a):
            off = s * _LANES
            return tuple(
                a[r] + buf[r, pl.ds(off, _LANES)] for r in range(_TILE_S))

        @pl.when(nxt < _NCH)
        def _refill():
            cb = pl.multiple_of(nxt * _W, 128)
            pltpu.async_copy(
                y_hbm.at[pl.ds(rows, _TILE_S), pl.ds(cb, _W)], buf, dsem)
        return inner

    zero16 = jnp.zeros((_LANES,), jnp.float32)
    for g in range(_RW // _TILE_S):
        rows = pl.multiple_of(base + g * _TILE_S, _TILE_S)
        pltpu.async_copy(
            y_hbm.at[pl.ds(rows, _TILE_S), pl.ds(0, _W)], buf_a, sem_a)
        pltpu.async_copy(
            y_hbm.at[pl.ds(rows, _TILE_S), pl.ds(_W, _W)], buf_b, sem_b)

        @pl.loop(0, _NCH, step=2, init_carry=(zero16,) * _TILE_S)
        def chunk_loop(c, accs):
            accs = _drain_accum_refill(buf_a, sem_a, accs, rows, c + 2)
            return _drain_accum_refill(buf_b, sem_b, accs, rows, c + 3)

        for r in range(_TILE_S):
            rsb_v[g * _TILE_S + r, :] = chunk_loop[r]

    col0.wait()
    for c in copies:
        c.wait()
    acc = jnp.zeros((_LANES,), jnp.float32)
    zeros16 = jnp.zeros((_LANES,), jnp.int32)
    for k in range(_RW // _LANES):
        t16 = tgt_v[pl.ds(k * _LANES, _LANES)]
        rows16 = iota16 + (k * _LANES)
        sub16 = lax.rem(rows16, _TILE_S)
        lanes16 = lax.rem(t16, _TILE_L)
        yt = plsc.load_gather(buf_t, [rows16, sub16, lanes16])
        y0 = plsc.load_gather(buf_0, [rows16, zeros16])
        rs = zero16
        for l in range(_LANES):
            li = jnp.full((_LANES,), l, jnp.int32)
            rs = rs + plsc.load_gather(rsb_v, [rows16, li])
        contrib = (_E_TERM
                   - _SVAL * (rs - y0 - yt)
                   - _CONF * yt)
        acc = acc + jnp.where(t16 != _PAD_IDX, contrib, 0.0)
    acc_v[...] = acc
    pltpu.sync_copy(acc_v, out_hbm.at[pl.ds(wid * _LANES, _LANES)])


def _sc_partials(y, target):
    mesh = plsc.VectorSubcoreMesh(core_axis_name="c", subcore_axis_name="s")
    fn = pl.kernel(
        _sc_body,
        out_type=jax.ShapeDtypeStruct((_NW * _LANES,), jnp.float32),
        mesh=mesh,
        compiler_params=pltpu.CompilerParams(needs_layout_passes=False),
        scratch_types=[
            pltpu.VMEM((_RW,), jnp.int32),
            pltpu.VMEM((_RW, _TILE_S, _TILE_L), jnp.float32),
            pltpu.VMEM((_RW, _TILE_L), jnp.float32),
            pltpu.VMEM((_RW, _LANES), jnp.float32),
            pltpu.VMEM((_TILE_S, _W), jnp.float32),
            pltpu.VMEM((_TILE_S, _W), jnp.float32),
            pltpu.VMEM((_LANES,), jnp.float32),
            pltpu.SemaphoreType.DMA,
            pltpu.SemaphoreType.DMA,
            pltpu.SemaphoreType.DMA,
            pltpu.SemaphoreType.DMA,
        ],
    )
    return fn(y, target)


def kernel(y, target):
    target = target.astype(jnp.int32)
    tc_scalar = _tc_masked_rowsum(y, target.reshape(_N_TOK, 1))
    sc_out = _sc_partials(y, target)
    return jnp.sum(sc_out) - _SVAL * tc_scalar[0, 0]


# speedup vs baseline: 1.0019x; 1.0019x over previous
import math

import jax
import jax.numpy as jnp
from jax import lax
from jax.experimental import pallas as pl
from jax.experimental.pallas import tpu as pltpu
from jax.experimental.pallas import tpu_sc as plsc

_VOCAB = 100000
_PAD_IDX = 0
_SMOOTH = 0.1
_CONF = 1.0 - _SMOOTH
_N_TOK = 2048
_SVAL = _SMOOTH / (_VOCAB - 2)
_E_TERM = (_VOCAB - 2) * _SVAL * math.log(_SVAL) + _CONF * math.log(_CONF)

_CS = 38400
_W = 1920
_NCH = _CS // _W

_RB = 256
_VB = 12800
_RGRID = _N_TOK // _RB
_VGRID = -(-_VOCAB // _VB)
_JGRID = _VGRID - _CS // _VB

_NW = 32
_RW = _N_TOK // _NW
_LANES = 16
_TILE_S = 8
_TILE_L = 128

_TAIL_BASE = (_VGRID - 1) * _VB
_TAIL_FULL = (_VOCAB - _TAIL_BASE) // 128
_TAIL_REM = _VOCAB - _TAIL_BASE - _TAIL_FULL * 128


def _psum_lanes(y_ref, nslices, init=None):
    acc = init
    for k in range(nslices):
        s = y_ref[:, k * 128:(k + 1) * 128]
        acc = s if acc is None else acc + s
    return acc


def _rowsum_body(y_ref, tgt_ref, out_ref, acc_ref):
    i = pl.program_id(0)
    j = pl.program_id(1)

    @pl.when(j == 0)
    def _init():
        acc_ref[...] = _psum_lanes(y_ref, _VB // 128)

    @pl.when(jnp.logical_and(j > 0, j < _JGRID - 1))
    def _acc():
        acc_ref[...] = _psum_lanes(y_ref, _VB // 128, acc_ref[...])

    @pl.when(j == _JGRID - 1)
    def _acc_tail():
        acc = _psum_lanes(y_ref, _TAIL_FULL, acc_ref[...])
        if _TAIL_REM:
            lanes = lax.broadcasted_iota(jnp.int32, (_RB, 128), 1)
            part = y_ref[:, _TAIL_FULL * 128:(_TAIL_FULL + 1) * 128]
            acc = acc + jnp.where(lanes < _TAIL_REM, part, 0.0)
        rs_row = jnp.sum(acc, axis=1, keepdims=True)
        masked = jnp.where(tgt_ref[...] != _PAD_IDX, rs_row, 0.0)
        partial = jnp.sum(masked)

        @pl.when(i == 0)
        def _first():
            out_ref[0, 0] = partial

        @pl.when(i > 0)
        def _rest():
            out_ref[0, 0] += partial


def _tc_masked_rowsum(y, target_2d):
    return pl.pallas_call(
        _rowsum_body,
        grid=(_RGRID, _JGRID),
        in_specs=[
            pl.BlockSpec((_RB, _VB), lambda i, j: (i, j + _CS // _VB)),
            pl.BlockSpec((_RB, 1), lambda i, j: (i, 0)),
        ],
        out_specs=pl.BlockSpec((1, 1), lambda i, j: (0, 0),
                               memory_space=pltpu.SMEM),
        out_shape=jax.ShapeDtypeStruct((1, 1), jnp.float32),
        scratch_shapes=[pltpu.VMEM((_RB, 128), jnp.float32)],
    )(y, target_2d)


def _sc_body(y_hbm, tgt_hbm, out_hbm,
             tgt_v, buf_t, buf_0, rsb_v, buf_a, buf_b, acc_v,
             sem, sem2, sem_a, sem_b):
    wid = lax.axis_index("s") * 2 + lax.axis_index("c")
    base = wid * _RW
    pltpu.sync_copy(tgt_hbm.at[pl.ds(base, _RW)], tgt_v)
    col0 = pltpu.async_copy(
        y_hbm.at[pl.ds(base, _RW), pl.ds(0, _TILE_L)], buf_0, sem2)
    iota16 = lax.iota(jnp.int32, _LANES)
    copies = []
    for r in range(_RW):
        t16 = tgt_v[pl.ds((r // _LANES) * _LANES, _LANES)]
        t = jnp.sum(jnp.where(iota16 == (r % _LANES), t16, 0), axis=0)
        cb = pl.multiple_of((t // _TILE_L) * _TILE_L, _TILE_L)
        rg = pl.multiple_of(base + (r // _TILE_S) * _TILE_S, _TILE_S)
        copies.append(pltpu.async_copy(
            y_hbm.at[pl.ds(rg, _TILE_S), pl.ds(cb, _TILE_L)],
            buf_t.at[r], sem))

    def _drain_accum_refill(buf, dsem, accs, rows, nxt):
        pltpu.make_async_copy(
            y_hbm.at[pl.ds(0, _TILE_S), pl.ds(0, _W)], buf, dsem).wait()

        @pl.loop(0, _W // _LANES, init_carry=accs, unroll=8)
        def inner(s, a):
            off = s * _LANES
            return tuple(
                a[r] + buf[r, pl.ds(off, _LANES)] for r in range(_TILE_S))

        @pl.when(nxt < _NCH)
        def _refill():
            cb = pl.multiple_of(nxt * _W, 128)
            pltpu.async_copy(
                y_hbm.at[pl.ds(rows, _TILE_S), pl.ds(cb, _W)], buf, dsem)
        return inner

    zero16 = jnp.zeros((_LANES,), jnp.float32)
    for g in range(_RW // _TILE_S):
        rows = pl.multiple_of(base + g * _TILE_S, _TILE_S)
        pltpu.async_copy(
            y_hbm.at[pl.ds(rows, _TILE_S), pl.ds(0, _W)], buf_a, sem_a)
        pltpu.async_copy(
            y_hbm.at[pl.ds(rows, _TILE_S), pl.ds(_W, _W)], buf_b, sem_b)

        @pl.loop(0, _NCH, step=2, init_carry=(zero16,) * _TILE_S)
        def chunk_loop(c, accs):
            accs = _drain_accum_refill(buf_a, sem_a, accs, rows, c + 2)
            return _drain_accum_refill(buf_b, sem_b, accs, rows, c + 3)

        for r in range(_TILE_S):
            rsb_v[g * _TILE_S + r, :] = chunk_loop[r]

    col0.wait()
    for c in copies:
        c.wait()
    acc = jnp.zeros((_LANES,), jnp.float32)
    zeros16 = jnp.zeros((_LANES,), jnp.int32)
    for k in range(_RW // _LANES):
        t16 = tgt_v[pl.ds(k * _LANES, _LANES)]
        rows16 = iota16 + (k * _LANES)
        sub16 = lax.rem(rows16, _TILE_S)
        lanes16 = lax.rem(t16, _TILE_L)
        yt = plsc.load_gather(buf_t, [rows16, sub16, lanes16])
        y0 = plsc.load_gather(buf_0, [rows16, zeros16])
        rs = zero16
        for l in range(_LANES):
            li = jnp.full((_LANES,), l, jnp.int32)
            rs = rs + plsc.load_gather(rsb_v, [rows16, li])
        contrib = (_E_TERM
                   - _SVAL * (rs - y0 - yt)
                   - _CONF * yt)
        acc = acc + jnp.where(t16 != _PAD_IDX, contrib, 0.0)
    acc_v[...] = acc
    pltpu.sync_copy(acc_v, out_hbm.at[pl.ds(wid * _LANES, _LANES)])


def _sc_partials(y, target):
    mesh = plsc.VectorSubcoreMesh(core_axis_name="c", subcore_axis_name="s")
    fn = pl.kernel(
        _sc_body,
        out_type=jax.ShapeDtypeStruct((_NW * _LANES,), jnp.float32),
        mesh=mesh,
        compiler_params=pltpu.CompilerParams(needs_layout_passes=False),
        scratch_types=[
            pltpu.VMEM((_RW,), jnp.int32),
            pltpu.VMEM((_RW, _TILE_S, _TILE_L), jnp.float32),
            pltpu.VMEM((_RW, _TILE_L), jnp.float32),
            pltpu.VMEM((_RW, _LANES), jnp.float32),
            pltpu.VMEM((_TILE_S, _W), jnp.float32),
            pltpu.VMEM((_TILE_S, _W), jnp.float32),
            pltpu.VMEM((_LANES,), jnp.float32),
            pltpu.SemaphoreType.DMA,
            pltpu.SemaphoreType.DMA,
            pltpu.SemaphoreType.DMA,
            pltpu.SemaphoreType.DMA,
        ],
    )
    return fn(y, target)


def kernel(y, target):
    target = target.astype(jnp.int32)
    sc_out = _sc_partials(y, target)
    tc_scalar = _tc_masked_rowsum(y, target.reshape(_N_TOK, 1))
    return jnp.sum(sc_out) - _SVAL * tc_scalar[0, 0]


# speedup vs baseline: 1.0028x; 1.0008x over previous
import math

import jax
import jax.numpy as jnp
from jax import lax
from jax.experimental import pallas as pl
from jax.experimental.pallas import tpu as pltpu
from jax.experimental.pallas import tpu_sc as plsc

_VOCAB = 100000
_PAD_IDX = 0
_SMOOTH = 0.1
_CONF = 1.0 - _SMOOTH
_N_TOK = 2048
_SVAL = _SMOOTH / (_VOCAB - 2)
_E_TERM = (_VOCAB - 2) * _SVAL * math.log(_SVAL) + _CONF * math.log(_CONF)

_CS = 38400
_W = 1920
_NCH = _CS // _W

_RB = 256
_VB = 12800
_RGRID = _N_TOK // _RB
_VGRID = -(-_VOCAB // _VB)
_JGRID = _VGRID - _CS // _VB

_NW = 32
_RW = _N_TOK // _NW
_LANES = 16
_TILE_S = 8
_TILE_L = 128

_TAIL_BASE = (_VGRID - 1) * _VB
_TAIL_FULL = (_VOCAB - _TAIL_BASE) // 128
_TAIL_REM = _VOCAB - _TAIL_BASE - _TAIL_FULL * 128


def _psum_lanes(y_ref, nslices, init=None):
    acc = init
    for k in range(nslices):
        s = y_ref[:, k * 128:(k + 1) * 128]
        acc = s if acc is None else acc + s
    return acc


def _rowsum_body(y_ref, tgt_ref, out_ref, acc_ref):
    i = pl.program_id(0)
    j = pl.program_id(1)

    @pl.when(j == 0)
    def _init():
        acc_ref[...] = _psum_lanes(y_ref, _VB // 128)

    @pl.when(jnp.logical_and(j > 0, j < _JGRID - 1))
    def _acc():
        acc_ref[...] = _psum_lanes(y_ref, _VB // 128, acc_ref[...])

    @pl.when(j == _JGRID - 1)
    def _acc_tail():
        acc = _psum_lanes(y_ref, _TAIL_FULL, acc_ref[...])
        if _TAIL_REM:
            lanes = lax.broadcasted_iota(jnp.int32, (_RB, 128), 1)
            part = y_ref[:, _TAIL_FULL * 128:(_TAIL_FULL + 1) * 128]
            acc = acc + jnp.where(lanes < _TAIL_REM, part, 0.0)
        rs_row = jnp.sum(acc, axis=1, keepdims=True)
        masked = jnp.where(tgt_ref[...] != _PAD_IDX, rs_row, 0.0)
        partial = jnp.sum(masked)

        @pl.when(i == 0)
        def _first():
            out_ref[0, 0] = partial

        @pl.when(i > 0)
        def _rest():
            out_ref[0, 0] += partial


def _tc_masked_rowsum(y, target_2d):
    return pl.pallas_call(
        _rowsum_body,
        grid=(_RGRID, _JGRID),
        in_specs=[
            pl.BlockSpec((_RB, _VB), lambda i, j: (i, j + _CS // _VB)),
            pl.BlockSpec((_RB, 1), lambda i, j: (i, 0)),
        ],
        out_specs=pl.BlockSpec((1, 1), lambda i, j: (0, 0),
                               memory_space=pltpu.SMEM),
        out_shape=jax.ShapeDtypeStruct((1, 1), jnp.float32),
        scratch_shapes=[pltpu.VMEM((_RB, 128), jnp.float32)],
    )(y, target_2d)


def _sc_body(y_hbm, tgt_hbm, out_hbm,
             tgt_v, buf_t, buf_0, rsb_v, buf_a, buf_b, acc_v,
             sem, sem2, sem_a, sem_b):
    wid = lax.axis_index("s") * 2 + lax.axis_index("c")
    base = wid * _RW
    pltpu.sync_copy(tgt_hbm.at[pl.ds(base, _RW)], tgt_v)
    col0 = pltpu.async_copy(
        y_hbm.at[pl.ds(base, _RW), pl.ds(0, _TILE_L)], buf_0, sem2)
    iota16 = lax.iota(jnp.int32, _LANES)
    copies = []
    for r in range(_RW):
        t16 = tgt_v[pl.ds((r // _LANES) * _LANES, _LANES)]
        t = jnp.sum(jnp.where(iota16 == (r % _LANES), t16, 0), axis=0)
        cb = pl.multiple_of((t // _TILE_L) * _TILE_L, _TILE_L)
        rg = pl.multiple_of(base + (r // _TILE_S) * _TILE_S, _TILE_S)
        copies.append(pltpu.async_copy(
            y_hbm.at[pl.ds(rg, _TILE_S), pl.ds(cb, _TILE_L)],
            buf_t.at[r], sem))

    def _drain_accum_refill(buf, dsem, accs, rows, nxt):
        pltpu.make_async_copy(
            y_hbm.at[pl.ds(0, _TILE_S), pl.ds(0, _W)], buf, dsem).wait()

        @pl.loop(0, _W // _LANES, init_carry=accs, unroll=8)
        def inner(s, a):
            off = s * _LANES
            return tuple(
                a[r] + buf[r, pl.ds(off, _LANES)] for r in range(_TILE_S))

        @pl.when(nxt < _NCH)
        def _refill():
            cb = pl.multiple_of(nxt * _W, 128)
            pltpu.async_copy(
                y_hbm.at[pl.ds(rows, _TILE_S), pl.ds(cb, _W)], buf, dsem)
        return inner

    zero16 = jnp.zeros((_LANES,), jnp.float32)
    for g in range(_RW // _TILE_S):
        rows = pl.multiple_of(base + g * _TILE_S, _TILE_S)
        pltpu.async_copy(
            y_hbm.at[pl.ds(rows, _TILE_S), pl.ds(0, _W)], buf_a, sem_a)
        pltpu.async_copy(
            y_hbm.at[pl.ds(rows, _TILE_S), pl.ds(_W, _W)], buf_b, sem_b)

        @pl.loop(0, _NCH, step=2, init_carry=(zero16,) * _TILE_S)
        def chunk_loop(c, accs):
            accs = _drain_accum_refill(buf_a, sem_a, accs, rows, c + 2)
            return _drain_accum_refill(buf_b, sem_b, accs, rows, c + 3)

        for r in range(_TILE_S):
            rsb_v[g * _TILE_S + r, :] = chunk_loop[r]

    col0.wait()
    for c in copies:
        c.wait()
    acc = jnp.zeros((_LANES,), jnp.float32)
    zeros16 = jnp.zeros((_LANES,), jnp.int32)
    for k in range(_RW // _LANES):
        t16 = tgt_v[pl.ds(k * _LANES, _LANES)]
        rows16 = iota16 + (k * _LANES)
        sub16 = lax.rem(rows16, _TILE_S)
        lanes16 = lax.rem(t16, _TILE_L)
        yt = plsc.load_gather(buf_t, [rows16, sub16, lanes16])
        y0 = plsc.load_gather(buf_0, [rows16, zeros16])
        rs = zero16
        for l in range(_LANES):
            li = jnp.full((_LANES,), l, jnp.int32)
            rs = rs + plsc.load_gather(rsb_v, [rows16, li])
        contrib = (_E_TERM
                   - _SVAL * (rs - y0 - yt)
                   - _CONF * yt)
        acc = acc + jnp.where(t16 != _PAD_IDX, contrib, 0.0)
    acc_v[...] = acc
    pltpu.sync_copy(acc_v, out_hbm.at[pl.ds(wid * _LANES, _LANES)])


def _sc_partials(y, target):
    mesh = plsc.VectorSubcoreMesh(core_axis_name="c", subcore_axis_name="s")
    fn = pl.kernel(
        _sc_body,
        out_type=jax.ShapeDtypeStruct((_NW * _LANES,), jnp.float32),
        mesh=mesh,
        compiler_params=pltpu.CompilerParams(
            needs_layout_passes=False, skip_device_barrier=True),
        scratch_types=[
            pltpu.VMEM((_RW,), jnp.int32),
            pltpu.VMEM((_RW, _TILE_S, _TILE_L), jnp.float32),
            pltpu.VMEM((_RW, _TILE_L), jnp.float32),
            pltpu.VMEM((_RW, _LANES), jnp.float32),
            pltpu.VMEM((_TILE_S, _W), jnp.float32),
            pltpu.VMEM((_TILE_S, _W), jnp.float32),
            pltpu.VMEM((_LANES,), jnp.float32),
            pltpu.SemaphoreType.DMA,
            pltpu.SemaphoreType.DMA,
            pltpu.SemaphoreType.DMA,
            pltpu.SemaphoreType.DMA,
        ],
    )
    return fn(y, target)


def kernel(y, target):
    target = target.astype(jnp.int32)
    sc_out = _sc_partials(y, target)
    tc_scalar = _tc_masked_rowsum(y, target.reshape(_N_TOK, 1))
    return jnp.sum(sc_out) - _SVAL * tc_scalar[0, 0]
